# baseline (speedup 1.0000x reference)
import functools

import jax
import jax.numpy as jnp
from jax import lax
from jax.experimental import pallas as pl
from jax.experimental.pallas import tpu as pltpu
from jax.experimental.pallas import tpu_sc as plsc

H = 224
W = 224
P = 128
MR = 4
NR = MR + 1
K = float((2.0 / (W - 1 + 1e-8)) * 0.5 * (W - 1))

NTILES = 32
BAND = H // NTILES
HALO = 24
SLAB = BAND + 2 * HALO
SLABW = SLAB * W
NPIX = BAND * W
LANES = 16
NCH = NPIX // LANES
ROWCH = W // LANES


def _prep_body(xp_ref, ori_ref, rs_ref, bank_ref, cos_ref, sin_ref, rbase_ref):
    xp = xp_ref[...]
    bank_ref[0] = xp[MR:MR + H, MR:MR + W]
    h = xp[:, MR:MR + W]
    for r in range(1, NR):
        h = h + xp[:, MR - r:MR - r + W] + xp[:, MR + r:MR + r + W]
        v = h[MR:MR + H, :]
        for d in range(1, r + 1):
            v = v + h[MR - d:MR - d + H, :] + h[MR + d:MR + d + H, :]
        bank_ref[r] = v * (1.0 / float((2 * r + 1) ** 2))
    ori = ori_ref[...]
    cos_ref[...] = jnp.cos(ori) * K
    sin_ref[...] = jnp.sin(ori) * K
    rs = rs_ref[...]
    rv = lax.broadcasted_iota(jnp.int32, (NR, P), 0).astype(jnp.float32)
    rbase_ref[...] = (jnp.sum(rs * rv, axis=0, keepdims=True)
                      * float(SLABW)).astype(jnp.int32)


_prep = pl.pallas_call(
    _prep_body,
    out_shape=[
        jax.ShapeDtypeStruct((NR, H, W), jnp.float32),
        jax.ShapeDtypeStruct((H, W), jnp.float32),
        jax.ShapeDtypeStruct((H, W), jnp.float32),
        jax.ShapeDtypeStruct((1, P), jnp.int32),
    ],
)


@functools.lru_cache(maxsize=1)
def _build_sc_sample():
  mesh = plsc.VectorSubcoreMesh(core_axis_name="c", subcore_axis_name="s")

  @functools.partial(
      pl.kernel,
      mesh=mesh,
      compiler_params=pltpu.CompilerParams(needs_layout_passes=False),
      out_type=jax.ShapeDtypeStruct((P * H * W,), jnp.float32),
      scratch_types=[
          pltpu.VMEM((NR * SLABW,), jnp.float32),
          pltpu.VMEM((NPIX,), jnp.float32),
          pltpu.VMEM((NPIX,), jnp.float32),
          pltpu.VMEM((P,), jnp.float32),
          pltpu.VMEM((P,), jnp.float32),
          pltpu.VMEM((P,), jnp.float32),
          pltpu.VMEM((P,), jnp.float32),
          pltpu.VMEM((P,), jnp.float32),
          pltpu.VMEM((P,), jnp.int32),
          pltpu.VMEM((2 * NPIX,), jnp.float32),
          pltpu.SemaphoreType.DMA,
          pltpu.SemaphoreType.DMA,
      ],
  )
  def _sc_sample(bank_hbm, cos_hbm, sin_hbm, rbase_hbm, ox1_hbm, oy1_hbm,
                 ox2_hbm, oy2_hbm, thr_hbm, out_hbm,
                 bank_v, cos_v, sin_v, ox1_v, oy1_v, ox2_v, oy2_v, thr_v,
                 rbase_v, out_v, sem0, sem1):
    wid = lax.axis_index("s") * 2 + lax.axis_index("c")
    band0 = wid * BAND
    start = jnp.clip(band0 - HALO, 0, H - SLAB)
    for img in range(NR):
        pltpu.sync_copy(bank_hbm.at[pl.ds(img * H * W + start * W, SLABW)],
                        bank_v.at[pl.ds(img * SLABW, SLABW)])
    pltpu.sync_copy(cos_hbm.at[pl.ds(band0 * W, NPIX)], cos_v)
    pltpu.sync_copy(sin_hbm.at[pl.ds(band0 * W, NPIX)], sin_v)
    pltpu.sync_copy(ox1_hbm, ox1_v)
    pltpu.sync_copy(oy1_hbm, oy1_v)
    pltpu.sync_copy(ox2_hbm, ox2_v)
    pltpu.sync_copy(oy2_hbm, oy2_v)
    pltpu.sync_copy(thr_hbm, thr_v)
    pltpu.sync_copy(rbase_hbm, rbase_v)

    lane = lax.iota(jnp.int32, LANES)
    lanefK = lane.astype(jnp.float32) * K
    startW = start * W
    band0f = (band0 * 1.0)

    def compute_pair(p, boff):
        pidx = jnp.full((LANES,), p, jnp.int32)
        ox1 = plsc.load_gather(ox1_v, [pidx])
        oy1 = plsc.load_gather(oy1_v, [pidx])
        ox2 = plsc.load_gather(ox2_v, [pidx])
        oy2 = plsc.load_gather(oy2_v, [pidx])
        thr = plsc.load_gather(thr_v, [pidx])
        radj = plsc.load_gather(rbase_v, [pidx]) - startW

        @plsc.parallel_loop(0, NCH, unroll=7)
        def chunk_body(c):
            row = lax.div(c, ROWCH)
            cx = lax.rem(c, ROWCH)
            coff = c * LANES
            cs = cos_v[pl.ds(coff, LANES)]
            sn = sin_v[pl.ds(coff, LANES)]
            byK = jnp.full((LANES,), (band0 + row).astype(jnp.float32) * K)
            bxK = jnp.full((LANES,), (cx * LANES).astype(jnp.float32) * K) \
                + lanefK

            def samp(oxv, oyv):
                ix = jnp.clip(bxK + (oxv * cs - oyv * sn), 0.0, float(W - 1))
                iy = jnp.clip(byK + (oxv * sn + oyv * cs), 0.0, float(H - 1))
                ix0 = ix.astype(jnp.int32)
                iy0 = iy.astype(jnp.int32)
                wx = ix - ix0.astype(jnp.float32)
                wy = iy - iy0.astype(jnp.float32)
                ix1 = jnp.minimum(ix0 + 1, W - 1)
                iy1 = jnp.minimum(iy0 + 1, H - 1)
                base0 = radj + iy0 * W
                base1 = radj + iy1 * W
                v00 = plsc.load_gather(bank_v, [base0 + ix0])
                v01 = plsc.load_gather(bank_v, [base0 + ix1])
                v10 = plsc.load_gather(bank_v, [base1 + ix0])
                v11 = plsc.load_gather(bank_v, [base1 + ix1])
                top = v00 + wx * (v01 - v00)
                bot = v10 + wx * (v11 - v10)
                return top + wy * (bot - top)

            s1 = samp(ox1, oy1)
            s2 = samp(ox2, oy2)
            out_v[pl.ds(boff + coff, LANES)] = s1 - s2 - thr

    def pair2_body(i, carry):
        for half, sem in ((0, sem0), (1, sem1)):
            p = i * 2 + half
            boff = half * NPIX

            @pl.when(i >= 1)
            def _wait():
                pltpu.make_async_copy(out_v.at[pl.ds(boff, NPIX)],
                                      out_hbm.at[pl.ds(0, NPIX)], sem).wait()

            compute_pair(p, boff)
            pltpu.make_async_copy(
                out_v.at[pl.ds(boff, NPIX)],
                out_hbm.at[pl.ds(p * (H * W) + band0 * W, NPIX)],
                sem).start()
        return carry

    lax.fori_loop(0, P // 2, pair2_body, 0)
    pltpu.make_async_copy(out_v.at[pl.ds(0, NPIX)],
                          out_hbm.at[pl.ds(0, NPIX)], sem0).wait()
    pltpu.make_async_copy(out_v.at[pl.ds(NPIX, NPIX)],
                          out_hbm.at[pl.ds(0, NPIX)], sem1).wait()

  return _sc_sample


def kernel(x, orientation, offset_x1, offset_x2, offset_y1, offset_y2,
           radius_select, box_kernel_bank, thresholds):
    del box_kernel_bank
    xp = jnp.pad(x.reshape(H, W), MR, mode="edge")
    bank, cosim, sinim, rbase = _prep(xp, orientation.reshape(H, W),
                                      radius_select)
    out_flat = _build_sc_sample()(bank.reshape(-1), cosim.reshape(-1),
                                  sinim.reshape(-1), rbase.reshape(-1),
                                  offset_x1, offset_y1, offset_x2,
                                  offset_y2, thresholds)
    return out_flat.reshape(1, P, H, W)


# speedup vs baseline: 1.5534x; 1.5534x over previous
import functools

import jax
import jax.numpy as jnp
from jax import lax
from jax.experimental import pallas as pl
from jax.experimental.pallas import tpu as pltpu
from jax.experimental.pallas import tpu_sc as plsc

H = 224
W = 224
P = 128
MR = 4
NR = MR + 1
K = float((2.0 / (W - 1 + 1e-8)) * 0.5 * (W - 1))

NTILES = 32
BAND = H // NTILES
HALO = 24
SLAB = BAND + 2 * HALO
SLABW = SLAB * W
NPIX = BAND * W
LANES = 16
NCH = NPIX // LANES
ROWCH = W // LANES


def _prep_body(xp_ref, ori_ref, rs_ref, bank_ref, cos_ref, sin_ref, rbase_ref):
    xp = xp_ref[...]
    bank_ref[0] = xp[MR:MR + H, MR:MR + W]
    h = xp[:, MR:MR + W]
    for r in range(1, NR):
        h = h + xp[:, MR - r:MR - r + W] + xp[:, MR + r:MR + r + W]
        v = h[MR:MR + H, :]
        for d in range(1, r + 1):
            v = v + h[MR - d:MR - d + H, :] + h[MR + d:MR + d + H, :]
        bank_ref[r] = v * (1.0 / float((2 * r + 1) ** 2))
    ori = ori_ref[...]
    cos_ref[...] = jnp.cos(ori) * K
    sin_ref[...] = jnp.sin(ori) * K
    rs = rs_ref[...]
    rv = lax.broadcasted_iota(jnp.int32, (NR, P), 0).astype(jnp.float32)
    rbase_ref[...] = (jnp.sum(rs * rv, axis=0, keepdims=True)
                      * float(SLABW)).astype(jnp.int32)


_prep = pl.pallas_call(
    _prep_body,
    out_shape=[
        jax.ShapeDtypeStruct((NR, H, W), jnp.float32),
        jax.ShapeDtypeStruct((H, W), jnp.float32),
        jax.ShapeDtypeStruct((H, W), jnp.float32),
        jax.ShapeDtypeStruct((1, P), jnp.int32),
    ],
)


@functools.lru_cache(maxsize=1)
def _build_sc_sample():
  mesh = plsc.VectorSubcoreMesh(core_axis_name="c", subcore_axis_name="s")

  @functools.partial(
      pl.kernel,
      mesh=mesh,
      compiler_params=pltpu.CompilerParams(needs_layout_passes=False),
      out_type=jax.ShapeDtypeStruct((P * H * W,), jnp.float32),
      scratch_types=[
          pltpu.VMEM((NR * SLABW,), jnp.float32),
          pltpu.VMEM((NPIX,), jnp.float32),
          pltpu.VMEM((NPIX,), jnp.float32),
          pltpu.VMEM((P,), jnp.float32),
          pltpu.VMEM((P,), jnp.float32),
          pltpu.VMEM((P,), jnp.float32),
          pltpu.VMEM((P,), jnp.float32),
          pltpu.VMEM((P,), jnp.float32),
          pltpu.VMEM((P,), jnp.int32),
          pltpu.VMEM((2 * NPIX,), jnp.float32),
          pltpu.SemaphoreType.DMA,
          pltpu.SemaphoreType.DMA,
      ],
  )
  def _sc_sample(bank_hbm, cos_hbm, sin_hbm, rbase_hbm, ox1_hbm, oy1_hbm,
                 ox2_hbm, oy2_hbm, thr_hbm, out_hbm,
                 bank_v, cos_v, sin_v, ox1_v, oy1_v, ox2_v, oy2_v, thr_v,
                 rbase_v, out_v, sem0, sem1):
    wid = lax.axis_index("s") * 2 + lax.axis_index("c")
    band0 = wid * BAND
    start = jnp.clip(band0 - HALO, 0, H - SLAB)
    for img in range(NR):
        pltpu.sync_copy(bank_hbm.at[pl.ds(img * H * W + start * W, SLABW)],
                        bank_v.at[pl.ds(img * SLABW, SLABW)])
    pltpu.sync_copy(cos_hbm.at[pl.ds(band0 * W, NPIX)], cos_v)
    pltpu.sync_copy(sin_hbm.at[pl.ds(band0 * W, NPIX)], sin_v)
    pltpu.sync_copy(ox1_hbm, ox1_v)
    pltpu.sync_copy(oy1_hbm, oy1_v)
    pltpu.sync_copy(ox2_hbm, ox2_v)
    pltpu.sync_copy(oy2_hbm, oy2_v)
    pltpu.sync_copy(thr_hbm, thr_v)
    pltpu.sync_copy(rbase_hbm, rbase_v)

    lane = lax.iota(jnp.int32, LANES)
    lanefK = lane.astype(jnp.float32) * K
    startW = start * W
    band0f = (band0 * 1.0)

    def compute_pair(p, boff):
        pidx = jnp.full((LANES,), p, jnp.int32)
        ox1 = plsc.load_gather(ox1_v, [pidx])
        oy1 = plsc.load_gather(oy1_v, [pidx])
        ox2 = plsc.load_gather(ox2_v, [pidx])
        oy2 = plsc.load_gather(oy2_v, [pidx])
        thr = plsc.load_gather(thr_v, [pidx])
        radj = plsc.load_gather(rbase_v, [pidx]) - startW

        @plsc.parallel_loop(0, NCH, unroll=2)
        def chunk_body(c):
            row = lax.div(c, ROWCH)
            cx = lax.rem(c, ROWCH)
            coff = c * LANES
            cs = cos_v[pl.ds(coff, LANES)]
            sn = sin_v[pl.ds(coff, LANES)]
            byK = jnp.full((LANES,), (band0 + row).astype(jnp.float32) * K)
            bxK = jnp.full((LANES,), (cx * LANES).astype(jnp.float32) * K) \
                + lanefK

            def samp(oxv, oyv):
                ix = jnp.clip(bxK + (oxv * cs - oyv * sn), 0.0, float(W - 1))
                iy = jnp.clip(byK + (oxv * sn + oyv * cs), 0.0, float(H - 1))
                ix0 = ix.astype(jnp.int32)
                iy0 = iy.astype(jnp.int32)
                wx = ix - ix0.astype(jnp.float32)
                wy = iy - iy0.astype(jnp.float32)
                ix1 = jnp.minimum(ix0 + 1, W - 1)
                iy1 = jnp.minimum(iy0 + 1, H - 1)
                base0 = radj + iy0 * W
                base1 = radj + iy1 * W
                v00 = plsc.load_gather(bank_v, [base0 + ix0])
                v01 = plsc.load_gather(bank_v, [base0 + ix1])
                v10 = plsc.load_gather(bank_v, [base1 + ix0])
                v11 = plsc.load_gather(bank_v, [base1 + ix1])
                top = v00 + wx * (v01 - v00)
                bot = v10 + wx * (v11 - v10)
                return top + wy * (bot - top)

            s1 = samp(ox1, oy1)
            s2 = samp(ox2, oy2)
            out_v[pl.ds(boff + coff, LANES)] = s1 - s2 - thr

    def pair2_body(i, carry):
        for half, sem in ((0, sem0), (1, sem1)):
            p = i * 2 + half
            boff = half * NPIX

            @pl.when(i >= 1)
            def _wait():
                pltpu.make_async_copy(out_v.at[pl.ds(boff, NPIX)],
                                      out_hbm.at[pl.ds(0, NPIX)], sem).wait()

            compute_pair(p, boff)
            pltpu.make_async_copy(
                out_v.at[pl.ds(boff, NPIX)],
                out_hbm.at[pl.ds(p * (H * W) + band0 * W, NPIX)],
                sem).start()
        return carry

    lax.fori_loop(0, P // 2, pair2_body, 0)
    pltpu.make_async_copy(out_v.at[pl.ds(0, NPIX)],
                          out_hbm.at[pl.ds(0, NPIX)], sem0).wait()
    pltpu.make_async_copy(out_v.at[pl.ds(NPIX, NPIX)],
                          out_hbm.at[pl.ds(0, NPIX)], sem1).wait()

  return _sc_sample


def kernel(x, orientation, offset_x1, offset_x2, offset_y1, offset_y2,
           radius_select, box_kernel_bank, thresholds):
    del box_kernel_bank
    xp = jnp.pad(x.reshape(H, W), MR, mode="edge")
    bank, cosim, sinim, rbase = _prep(xp, orientation.reshape(H, W),
                                      radius_select)
    out_flat = _build_sc_sample()(bank.reshape(-1), cosim.reshape(-1),
                                  sinim.reshape(-1), rbase.reshape(-1),
                                  offset_x1, offset_y1, offset_x2,
                                  offset_y2, thresholds)
    return out_flat.reshape(1, P, H, W)


# speedup vs baseline: 1.7617x; 1.1341x over previous
import functools

import jax
import jax.numpy as jnp
from jax import lax
from jax.experimental import pallas as pl
from jax.experimental.pallas import tpu as pltpu
from jax.experimental.pallas import tpu_sc as plsc

H = 224
W = 224
P = 128
MR = 4
NR = MR + 1
K = float((2.0 / (W - 1 + 1e-8)) * 0.5 * (W - 1))

NTILES = 32
BAND = H // NTILES
HALO = 24
SLAB = BAND + 2 * HALO
SLABW = SLAB * W
NPIX = BAND * W
LANES = 16
NCH = NPIX // LANES
ROWCH = W // LANES


def _prep_body(xp_ref, ori_ref, rs_ref, bank_ref, cos_ref, sin_ref, rbase_ref):
    xp = xp_ref[...]
    bank_ref[0] = xp[MR:MR + H, MR:MR + W]
    h = xp[:, MR:MR + W]
    for r in range(1, NR):
        h = h + xp[:, MR - r:MR - r + W] + xp[:, MR + r:MR + r + W]
        v = h[MR:MR + H, :]
        for d in range(1, r + 1):
            v = v + h[MR - d:MR - d + H, :] + h[MR + d:MR + d + H, :]
        bank_ref[r] = v * (1.0 / float((2 * r + 1) ** 2))
    ori = ori_ref[...]
    cos_ref[...] = jnp.cos(ori) * K
    sin_ref[...] = jnp.sin(ori) * K
    rs = rs_ref[...]
    rv = lax.broadcasted_iota(jnp.int32, (NR, P), 0).astype(jnp.float32)
    rbase_ref[...] = (jnp.sum(rs * rv, axis=0, keepdims=True)
                      * float(SLABW)).astype(jnp.int32)


_prep = pl.pallas_call(
    _prep_body,
    out_shape=[
        jax.ShapeDtypeStruct((NR, H, W), jnp.float32),
        jax.ShapeDtypeStruct((H, W), jnp.float32),
        jax.ShapeDtypeStruct((H, W), jnp.float32),
        jax.ShapeDtypeStruct((1, P), jnp.int32),
    ],
)


@functools.lru_cache(maxsize=1)
def _build_sc_sample():
  mesh = plsc.VectorSubcoreMesh(core_axis_name="c", subcore_axis_name="s")

  @functools.partial(
      pl.kernel,
      mesh=mesh,
      compiler_params=pltpu.CompilerParams(needs_layout_passes=False),
      out_type=jax.ShapeDtypeStruct((P * H * W,), jnp.float32),
      scratch_types=[
          pltpu.VMEM((NR * SLABW + 256,), jnp.float32),
          pltpu.VMEM((NPIX,), jnp.float32),
          pltpu.VMEM((NPIX,), jnp.float32),
          pltpu.VMEM((P,), jnp.float32),
          pltpu.VMEM((P,), jnp.float32),
          pltpu.VMEM((P,), jnp.float32),
          pltpu.VMEM((P,), jnp.float32),
          pltpu.VMEM((P,), jnp.float32),
          pltpu.VMEM((P,), jnp.int32),
          pltpu.VMEM((2 * NPIX,), jnp.float32),
          pltpu.SemaphoreType.DMA,
          pltpu.SemaphoreType.DMA,
      ],
  )
  def _sc_sample(bank_hbm, cos_hbm, sin_hbm, rbase_hbm, ox1_hbm, oy1_hbm,
                 ox2_hbm, oy2_hbm, thr_hbm, out_hbm,
                 bank_v, cos_v, sin_v, ox1_v, oy1_v, ox2_v, oy2_v, thr_v,
                 rbase_v, out_v, sem0, sem1):
    wid = lax.axis_index("s") * 2 + lax.axis_index("c")
    band0 = wid * BAND
    start = jnp.clip(band0 - HALO, 0, H - SLAB)
    for img in range(NR):
        pltpu.sync_copy(bank_hbm.at[pl.ds(img * H * W + start * W, SLABW)],
                        bank_v.at[pl.ds(img * SLABW, SLABW)])
    pltpu.sync_copy(cos_hbm.at[pl.ds(band0 * W, NPIX)], cos_v)
    pltpu.sync_copy(sin_hbm.at[pl.ds(band0 * W, NPIX)], sin_v)
    pltpu.sync_copy(ox1_hbm, ox1_v)
    pltpu.sync_copy(oy1_hbm, oy1_v)
    pltpu.sync_copy(ox2_hbm, ox2_v)
    pltpu.sync_copy(oy2_hbm, oy2_v)
    pltpu.sync_copy(thr_hbm, thr_v)
    pltpu.sync_copy(rbase_hbm, rbase_v)

    zeros16 = jnp.zeros((LANES,), jnp.float32)
    for z in range(16):
        out_idx = NR * SLABW + z * LANES
        bank_v[pl.ds(out_idx, LANES)] = zeros16

    lane = lax.iota(jnp.int32, LANES)
    lanefK = lane.astype(jnp.float32) * K
    startW = start * W

    def compute_pair(p, boff):
        pidx = jnp.full((LANES,), p, jnp.int32)
        ox1 = plsc.load_gather(ox1_v, [pidx])
        oy1 = plsc.load_gather(oy1_v, [pidx])
        ox2 = plsc.load_gather(ox2_v, [pidx])
        oy2 = plsc.load_gather(oy2_v, [pidx])
        thr = plsc.load_gather(thr_v, [pidx])
        radj = plsc.load_gather(rbase_v, [pidx]) - startW

        @plsc.parallel_loop(0, NCH, unroll=2)
        def chunk_body(c):
            row = lax.div(c, ROWCH)
            cx = lax.rem(c, ROWCH)
            coff = c * LANES
            cs = cos_v[pl.ds(coff, LANES)]
            sn = sin_v[pl.ds(coff, LANES)]
            byK = jnp.full((LANES,), (band0 + row).astype(jnp.float32) * K)
            bxK = jnp.full((LANES,), (cx * LANES).astype(jnp.float32) * K) \
                + lanefK

            def samp(oxv, oyv):
                ix = jnp.clip(bxK + (oxv * cs - oyv * sn), 0.0, float(W - 1))
                iy = jnp.clip(byK + (oxv * sn + oyv * cs), 0.0, float(H - 1))
                ix0 = ix.astype(jnp.int32)
                iy0 = iy.astype(jnp.int32)
                wx = ix - ix0.astype(jnp.float32)
                wy = iy - iy0.astype(jnp.float32)
                idx00 = radj + iy0 * W + ix0
                v00 = plsc.load_gather(bank_v, [idx00])
                v01 = plsc.load_gather(bank_v, [idx00 + 1])
                v10 = plsc.load_gather(bank_v, [idx00 + W])
                v11 = plsc.load_gather(bank_v, [idx00 + (W + 1)])
                top = v00 + wx * (v01 - v00)
                bot = v10 + wx * (v11 - v10)
                return top + wy * (bot - top)

            s1 = samp(ox1, oy1)
            s2 = samp(ox2, oy2)
            out_v[pl.ds(boff + coff, LANES)] = s1 - s2 - thr

    def pair2_body(i, carry):
        for half, sem in ((0, sem0), (1, sem1)):
            p = i * 2 + half
            boff = half * NPIX

            @pl.when(i >= 1)
            def _wait():
                pltpu.make_async_copy(out_v.at[pl.ds(boff, NPIX)],
                                      out_hbm.at[pl.ds(0, NPIX)], sem).wait()

            compute_pair(p, boff)
            pltpu.make_async_copy(
                out_v.at[pl.ds(boff, NPIX)],
                out_hbm.at[pl.ds(p * (H * W) + band0 * W, NPIX)],
                sem).start()
        return carry

    lax.fori_loop(0, P // 2, pair2_body, 0)
    pltpu.make_async_copy(out_v.at[pl.ds(0, NPIX)],
                          out_hbm.at[pl.ds(0, NPIX)], sem0).wait()
    pltpu.make_async_copy(out_v.at[pl.ds(NPIX, NPIX)],
                          out_hbm.at[pl.ds(0, NPIX)], sem1).wait()

  return _sc_sample


def kernel(x, orientation, offset_x1, offset_x2, offset_y1, offset_y2,
           radius_select, box_kernel_bank, thresholds):
    del box_kernel_bank
    xp = jnp.pad(x.reshape(H, W), MR, mode="edge")
    bank, cosim, sinim, rbase = _prep(xp, orientation.reshape(H, W),
                                      radius_select)
    out_flat = _build_sc_sample()(bank.reshape(-1), cosim.reshape(-1),
                                  sinim.reshape(-1), rbase.reshape(-1),
                                  offset_x1, offset_y1, offset_x2,
                                  offset_y2, thresholds)
    return out_flat.reshape(1, P, H, W)
